# baseline (speedup 1.0000x reference)
import functools

import jax
import jax.numpy as jnp
from jax.experimental import pallas as pl
from jax.experimental.pallas import tpu as pltpu

_NUM_TABLES = 26
_NUM_PAIRS = _NUM_TABLES // 2
_BATCH = 4096
_K_HASH = 128
_EMB_DIM = 64
_M_HASH = 1000000
_BT = 1024
_ZW = 1792


def _fused_body(xd, xi, ha, hb, w1, b1, w2bd, b2c,
                bw1, bb1, bw2, bb2, bw3, bb3,
                tw1, tb1, tw2, tb2, tw3t, tb3, out, zbuf):
    f32 = jnp.float32
    dot = functools.partial(jnp.dot, preferred_element_type=f32)

    h = jnp.maximum(dot(xd[...], bw1[...]) + bb1[...], 0.0)
    h = jnp.maximum(dot(h, bw2[...]) + bb2[...], 0.0)
    h = jnp.maximum(dot(h, bw3[...]) + bb3[...], 0.0)
    zbuf[:, _ZW - 128:_ZW] = jnp.concatenate(
        [h, jnp.zeros((h.shape[0], _EMB_DIM), f32)], axis=1)

    for p in range(_NUM_PAIRS):
        e1s = []
        for j in (0, 1):
            t = 2 * p + j
            idx = xi[t, :].astype(jnp.uint32)
            a = ha[t, :].astype(jnp.uint32)
            b = hb[t, :].astype(jnp.uint32)
            hv = (idx[:, None] * a[None, :] + b[None, :]) % jnp.uint32(_M_HASH)
            hvf = hv.astype(f32)
            e1s.append(jnp.maximum(dot(hvf, w1[t]) + b1[t, :][None, :], 0.0))
        e1cat = jnp.concatenate(e1s, axis=1)
        embc = dot(e1cat, w2bd[p]) + b2c[p, :][None, :]
        zbuf[:, 128 * p:128 * (p + 1)] = embc

    z = jnp.maximum(dot(zbuf[...], tw1[...]) + tb1[...], 0.0)
    z = jnp.maximum(dot(z, tw2[...]) + tb2[...], 0.0)
    o = jnp.sum(z * tw3t[...], axis=1, keepdims=True) + tb3[...]
    out[...] = jax.nn.sigmoid(o)


def _make_call(interpret=False):
    bt = _BT
    grid = (_BATCH // bt,)

    def batch_spec(shape):
        return pl.BlockSpec(shape, lambda i: (i, 0))

    def const_spec(shape):
        nd = len(shape)
        if nd == 2:
            return pl.BlockSpec(shape, lambda i: (0, 0))
        return pl.BlockSpec(shape, lambda i: (0, 0, 0))

    in_specs = [
        batch_spec((bt, 13)),
        pl.BlockSpec((_NUM_TABLES, bt), lambda i: (0, i)),
        const_spec((_NUM_TABLES, _K_HASH)),
        const_spec((_NUM_TABLES, _K_HASH)),
        const_spec((_NUM_TABLES, _K_HASH, 256)),
        const_spec((_NUM_TABLES, 256)),
        const_spec((_NUM_PAIRS, 512, 128)),
        const_spec((_NUM_PAIRS, 128)),
        const_spec((13, 512)),
        const_spec((1, 512)),
        const_spec((512, 256)),
        const_spec((1, 256)),
        const_spec((256, 64)),
        const_spec((1, 64)),
        const_spec((_ZW, 512)),
        const_spec((1, 512)),
        const_spec((512, 256)),
        const_spec((1, 256)),
        const_spec((1, 256)),
        const_spec((1, 1)),
    ]
    return pl.pallas_call(
        _fused_body,
        grid=grid,
        in_specs=in_specs,
        out_specs=pl.BlockSpec((bt, 1), lambda i: (i, 0)),
        out_shape=jax.ShapeDtypeStruct((_BATCH, 1), jnp.float32),
        scratch_shapes=[pltpu.VMEM((bt, _ZW), jnp.float32)],
        interpret=interpret,
    )


def kernel(x_dense, x_indices, hash_a, hash_b, dec_W1, dec_b1, dec_W2,
           dec_b2, bot_W1, bot_b1, bot_W2, bot_b2, bot_W3, bot_b3,
           top_W1, top_b1, top_W2, top_b2, top_W3, top_b3):
    f32 = jnp.float32
    scale = f32(2.0 / (_M_HASH - 1))
    w1 = dec_W1 * scale
    b1 = dec_b1 - dec_W1.sum(axis=1)
    w2bd = jnp.zeros((_NUM_PAIRS, 512, 128), f32)
    w2bd = w2bd.at[:, :256, :64].set(dec_W2[0::2])
    w2bd = w2bd.at[:, 256:, 64:].set(dec_W2[1::2])
    b2c = dec_b2.reshape(_NUM_PAIRS, 128)
    tw1 = jnp.concatenate(
        [top_W1[_EMB_DIM:], top_W1[:_EMB_DIM],
         jnp.zeros((_EMB_DIM, 512), f32)], axis=0)

    call = _make_call()
    return call(
        x_dense, x_indices, hash_a, hash_b,
        w1, b1, w2bd, b2c,
        bot_W1, bot_b1.reshape(1, -1),
        bot_W2, bot_b2.reshape(1, -1),
        bot_W3, bot_b3.reshape(1, -1),
        tw1, top_b1.reshape(1, -1),
        top_W2, top_b2.reshape(1, -1),
        top_W3.reshape(1, -1), top_b3.reshape(1, -1),
    )


# speedup vs baseline: 1.7157x; 1.7157x over previous
import functools

import jax
import jax.numpy as jnp
from jax.experimental import pallas as pl
from jax.experimental.pallas import tpu as pltpu

_NUM_TABLES = 26
_NUM_PAIRS = _NUM_TABLES // 2
_BATCH = 4096
_K_HASH = 128
_EMB_DIM = 64
_M_HASH = 1000000
_BT = 1024
_ZW = 1792


def _fused_body(xd, xi, ha, hb, w1, b1, w2, b2c,
                bw1, bb1, bw2, bb2, bw3, bb3,
                tw1, tb1, tw2, tb2, tw3t, tb3, out,
                zbuf, w1s, b1s, w2bd, tw1s):
    f32 = jnp.float32
    dot = functools.partial(jnp.dot, preferred_element_type=f32)
    scale = f32(2.0 / (_M_HASH - 1))

    @pl.when(pl.program_id(0) == 0)
    def _prep():
        w1s[...] = w1[...] * scale
        b1s[...] = b1[...] - jnp.sum(w1[...], axis=1)
        w2bd[...] = jnp.zeros((_NUM_PAIRS, 512, 128), f32)
        for p in range(_NUM_PAIRS):
            w2bd[p, 0:256, 0:64] = w2[2 * p]
            w2bd[p, 256:512, 64:128] = w2[2 * p + 1]
        tw1s[0:_NUM_PAIRS * 128, :] = tw1[_EMB_DIM:, :]
        tw1s[_NUM_PAIRS * 128:_NUM_PAIRS * 128 + _EMB_DIM, :] = tw1[0:_EMB_DIM, :]
        tw1s[_ZW - _EMB_DIM:_ZW, :] = jnp.zeros((_EMB_DIM, 512), f32)
        zbuf[:, _ZW - _EMB_DIM:_ZW] = jnp.zeros((_BT, _EMB_DIM), f32)

    h = jnp.maximum(dot(xd[...], bw1[...]) + bb1[...], 0.0)
    h = jnp.maximum(dot(h, bw2[...]) + bb2[...], 0.0)
    h = jnp.maximum(dot(h, bw3[...]) + bb3[...], 0.0)
    zbuf[:, _ZW - 128:_ZW - _EMB_DIM] = h

    for p in range(_NUM_PAIRS):
        e1s = []
        for j in (0, 1):
            t = 2 * p + j
            idx = xi[t, :].astype(jnp.uint32)
            a = ha[t, :].astype(jnp.uint32)
            b = hb[t, :].astype(jnp.uint32)
            hv = (idx[:, None] * a[None, :] + b[None, :]) % jnp.uint32(_M_HASH)
            hvf = hv.astype(f32)
            e1s.append(jnp.maximum(dot(hvf, w1s[t]) + b1s[t, :][None, :], 0.0))
        e1cat = jnp.concatenate(e1s, axis=1)
        embc = dot(e1cat, w2bd[p]) + b2c[p, :][None, :]
        zbuf[:, 128 * p:128 * (p + 1)] = embc

    z = jnp.maximum(dot(zbuf[...], tw1s[...]) + tb1[...], 0.0)
    z = jnp.maximum(dot(z, tw2[...]) + tb2[...], 0.0)
    o = jnp.sum(z * tw3t[...], axis=1, keepdims=True) + tb3[...]
    out[...] = jax.nn.sigmoid(o)


def _make_call(interpret=False):
    bt = _BT
    grid = (_BATCH // bt,)

    def batch_spec(shape):
        return pl.BlockSpec(shape, lambda i: (i, 0))

    def const_spec(shape):
        nd = len(shape)
        if nd == 2:
            return pl.BlockSpec(shape, lambda i: (0, 0))
        return pl.BlockSpec(shape, lambda i: (0, 0, 0))

    in_specs = [
        batch_spec((bt, 13)),
        pl.BlockSpec((_NUM_TABLES, bt), lambda i: (0, i)),
        const_spec((_NUM_TABLES, _K_HASH)),
        const_spec((_NUM_TABLES, _K_HASH)),
        const_spec((_NUM_TABLES, _K_HASH, 256)),
        const_spec((_NUM_TABLES, 256)),
        const_spec((_NUM_TABLES, 256, _EMB_DIM)),
        const_spec((_NUM_PAIRS, 128)),
        const_spec((13, 512)),
        const_spec((1, 512)),
        const_spec((512, 256)),
        const_spec((1, 256)),
        const_spec((256, 64)),
        const_spec((1, 64)),
        const_spec((27 * _EMB_DIM, 512)),
        const_spec((1, 512)),
        const_spec((512, 256)),
        const_spec((1, 256)),
        const_spec((1, 256)),
        const_spec((1, 1)),
    ]
    return pl.pallas_call(
        _fused_body,
        grid=grid,
        in_specs=in_specs,
        out_specs=pl.BlockSpec((bt, 1), lambda i: (i, 0)),
        out_shape=jax.ShapeDtypeStruct((_BATCH, 1), jnp.float32),
        scratch_shapes=[
            pltpu.VMEM((bt, _ZW), jnp.float32),
            pltpu.VMEM((_NUM_TABLES, _K_HASH, 256), jnp.float32),
            pltpu.VMEM((_NUM_TABLES, 256), jnp.float32),
            pltpu.VMEM((_NUM_PAIRS, 512, 128), jnp.float32),
            pltpu.VMEM((_ZW, 512), jnp.float32),
        ],
        interpret=interpret,
    )


def kernel(x_dense, x_indices, hash_a, hash_b, dec_W1, dec_b1, dec_W2,
           dec_b2, bot_W1, bot_b1, bot_W2, bot_b2, bot_W3, bot_b3,
           top_W1, top_b1, top_W2, top_b2, top_W3, top_b3):
    call = _make_call()
    return call(
        x_dense, x_indices, hash_a, hash_b,
        dec_W1, dec_b1, dec_W2, dec_b2.reshape(_NUM_PAIRS, 128),
        bot_W1, bot_b1.reshape(1, -1),
        bot_W2, bot_b2.reshape(1, -1),
        bot_W3, bot_b3.reshape(1, -1),
        top_W1, top_b1.reshape(1, -1),
        top_W2, top_b2.reshape(1, -1),
        top_W3.reshape(1, -1), top_b3.reshape(1, -1),
    )


# speedup vs baseline: 1.8302x; 1.0667x over previous
import functools

import jax
import jax.numpy as jnp
from jax.experimental import pallas as pl
from jax.experimental.pallas import tpu as pltpu

_NUM_TABLES = 26
_NUM_PAIRS = _NUM_TABLES // 2
_BATCH = 4096
_K_HASH = 128
_EMB_DIM = 64
_M_HASH = 1000000
_BT = 4096
_ZW = 1792


def _fused_body(xd, xi, ha, hb, w1, b1, w2, b2c,
                bw1, bb1, bw2, bb2, bw3, bb3,
                tw1, tb1, tw2, tb2, tw3t, tb3, out,
                zbuf, w1s, b1s, w2bd, tw1s):
    f32 = jnp.float32
    dot = functools.partial(jnp.dot, preferred_element_type=f32)
    scale = f32(2.0 / (_M_HASH - 1))

    @pl.when(pl.program_id(0) == 0)
    def _prep():
        w1s[...] = w1[...] * scale
        b1s[...] = b1[...] - jnp.sum(w1[...], axis=1)
        w2bd[...] = jnp.zeros((_NUM_PAIRS, 512, 128), f32)
        for p in range(_NUM_PAIRS):
            w2bd[p, 0:256, 0:64] = w2[2 * p]
            w2bd[p, 256:512, 64:128] = w2[2 * p + 1]
        tw1s[0:_NUM_PAIRS * 128, :] = tw1[_EMB_DIM:, :]
        tw1s[_NUM_PAIRS * 128:_NUM_PAIRS * 128 + _EMB_DIM, :] = tw1[0:_EMB_DIM, :]
        tw1s[_ZW - _EMB_DIM:_ZW, :] = jnp.zeros((_EMB_DIM, 512), f32)
        zbuf[:, _ZW - _EMB_DIM:_ZW] = jnp.zeros((_BT, _EMB_DIM), f32)

    h = jnp.maximum(dot(xd[...], bw1[...]) + bb1[...], 0.0)
    h = jnp.maximum(dot(h, bw2[...]) + bb2[...], 0.0)
    h = jnp.maximum(dot(h, bw3[...]) + bb3[...], 0.0)
    zbuf[:, _ZW - 128:_ZW - _EMB_DIM] = h

    for p in range(_NUM_PAIRS):
        e1s = []
        for j in (0, 1):
            t = 2 * p + j
            idx = xi[t, :].astype(jnp.uint32)
            a = ha[t, :].astype(jnp.uint32)
            b = hb[t, :].astype(jnp.uint32)
            hv = (idx[:, None] * a[None, :] + b[None, :]) % jnp.uint32(_M_HASH)
            hvf = hv.astype(jnp.int32).astype(f32)
            e1s.append(jnp.maximum(dot(hvf, w1s[t]) + b1s[t, :][None, :], 0.0))
        e1cat = jnp.concatenate(e1s, axis=1)
        embc = dot(e1cat, w2bd[p]) + b2c[p, :][None, :]
        zbuf[:, 128 * p:128 * (p + 1)] = embc

    z = jnp.maximum(dot(zbuf[...], tw1s[...]) + tb1[...], 0.0)
    z = jnp.maximum(dot(z, tw2[...]) + tb2[...], 0.0)
    o = jnp.sum(z * tw3t[...], axis=1, keepdims=True) + tb3[...]
    out[...] = jax.nn.sigmoid(o)


def _make_call(interpret=False):
    bt = _BT
    grid = (_BATCH // bt,)

    def batch_spec(shape):
        return pl.BlockSpec(shape, lambda i: (i, 0))

    def const_spec(shape):
        nd = len(shape)
        if nd == 2:
            return pl.BlockSpec(shape, lambda i: (0, 0))
        return pl.BlockSpec(shape, lambda i: (0, 0, 0))

    in_specs = [
        batch_spec((bt, 13)),
        pl.BlockSpec((_NUM_TABLES, bt), lambda i: (0, i)),
        const_spec((_NUM_TABLES, _K_HASH)),
        const_spec((_NUM_TABLES, _K_HASH)),
        const_spec((_NUM_TABLES, _K_HASH, 256)),
        const_spec((_NUM_TABLES, 256)),
        const_spec((_NUM_TABLES, 256, _EMB_DIM)),
        const_spec((_NUM_PAIRS, 128)),
        const_spec((13, 512)),
        const_spec((1, 512)),
        const_spec((512, 256)),
        const_spec((1, 256)),
        const_spec((256, 64)),
        const_spec((1, 64)),
        const_spec((27 * _EMB_DIM, 512)),
        const_spec((1, 512)),
        const_spec((512, 256)),
        const_spec((1, 256)),
        const_spec((1, 256)),
        const_spec((1, 1)),
    ]
    return pl.pallas_call(
        _fused_body,
        grid=grid,
        in_specs=in_specs,
        out_specs=pl.BlockSpec((bt, 1), lambda i: (i, 0)),
        out_shape=jax.ShapeDtypeStruct((_BATCH, 1), jnp.float32),
        scratch_shapes=[
            pltpu.VMEM((bt, _ZW), jnp.float32),
            pltpu.VMEM((_NUM_TABLES, _K_HASH, 256), jnp.float32),
            pltpu.VMEM((_NUM_TABLES, 256), jnp.float32),
            pltpu.VMEM((_NUM_PAIRS, 512, 128), jnp.float32),
            pltpu.VMEM((_ZW, 512), jnp.float32),
        ],
        interpret=interpret,
    )


def kernel(x_dense, x_indices, hash_a, hash_b, dec_W1, dec_b1, dec_W2,
           dec_b2, bot_W1, bot_b1, bot_W2, bot_b2, bot_W3, bot_b3,
           top_W1, top_b1, top_W2, top_b2, top_W3, top_b3):
    call = _make_call()
    return call(
        x_dense, x_indices, hash_a, hash_b,
        dec_W1, dec_b1, dec_W2, dec_b2.reshape(_NUM_PAIRS, 128),
        bot_W1, bot_b1.reshape(1, -1),
        bot_W2, bot_b2.reshape(1, -1),
        bot_W3, bot_b3.reshape(1, -1),
        top_W1, top_b1.reshape(1, -1),
        top_W2, top_b2.reshape(1, -1),
        top_W3.reshape(1, -1), top_b3.reshape(1, -1),
    )


# speedup vs baseline: 1.9501x; 1.0655x over previous
import functools

import jax
import jax.numpy as jnp
from jax.experimental import pallas as pl
from jax.experimental.pallas import tpu as pltpu

_NUM_TABLES = 26
_NUM_PAIRS = _NUM_TABLES // 2
_BATCH = 4096
_K_HASH = 128
_EMB_DIM = 64
_M_HASH = 1000000
_BT = 2048
_ZW = 1792


def _fused_body(xd, xi, ha, hb, w1, b1, w2, b2c,
                bw1, bb1, bw2, bb2, bw3, bb3,
                tw1, tb1, tw2, tb2, tw3t, tb3, out,
                zbuf, w1s, b1s, w2bd, tw1s):
    f32 = jnp.float32
    dot = functools.partial(jnp.dot, preferred_element_type=f32)
    scale = f32(2.0 / (_M_HASH - 1))

    @pl.when(pl.program_id(0) == 0)
    def _prep():
        w1s[...] = w1[...] * scale
        b1s[...] = b1[...] - jnp.sum(w1[...], axis=1)
        w2bd[...] = jnp.zeros((_NUM_PAIRS, 512, 128), f32)
        for p in range(_NUM_PAIRS):
            w2bd[p, 0:256, 0:64] = w2[2 * p]
            w2bd[p, 256:512, 64:128] = w2[2 * p + 1]
        tw1s[0:_NUM_PAIRS * 128, :] = tw1[_EMB_DIM:, :]
        tw1s[_NUM_PAIRS * 128:_NUM_PAIRS * 128 + _EMB_DIM, :] = tw1[0:_EMB_DIM, :]
        tw1s[_ZW - _EMB_DIM:_ZW, :] = jnp.zeros((_EMB_DIM, 512), f32)
        zbuf[:, _ZW - _EMB_DIM:_ZW] = jnp.zeros((_BT, _EMB_DIM), f32)

    h = jnp.maximum(dot(xd[...], bw1[...]) + bb1[...], 0.0)
    h = jnp.maximum(dot(h, bw2[...]) + bb2[...], 0.0)
    h = jnp.maximum(dot(h, bw3[...]) + bb3[...], 0.0)
    zbuf[:, _ZW - 128:_ZW - _EMB_DIM] = h

    for p in range(_NUM_PAIRS):
        e1s = []
        for j in (0, 1):
            t = 2 * p + j
            idx = xi[t, :].astype(jnp.uint32)
            a = ha[t, :].astype(jnp.uint32)
            b = hb[t, :].astype(jnp.uint32)
            hv = (idx[:, None] * a[None, :] + b[None, :]) % jnp.uint32(_M_HASH)
            hvf = hv.astype(jnp.int32).astype(f32)
            e1s.append(jnp.maximum(dot(hvf, w1s[t]) + b1s[t, :][None, :], 0.0))
        e1cat = jnp.concatenate(e1s, axis=1)
        embc = dot(e1cat, w2bd[p]) + b2c[p, :][None, :]
        zbuf[:, 128 * p:128 * (p + 1)] = embc

    z = jnp.maximum(dot(zbuf[...], tw1s[...]) + tb1[...], 0.0)
    z = jnp.maximum(dot(z, tw2[...]) + tb2[...], 0.0)
    o = jnp.sum(z * tw3t[...], axis=1, keepdims=True) + tb3[...]
    out[...] = jax.nn.sigmoid(o)


def _make_call(interpret=False):
    bt = _BT
    grid = (_BATCH // bt,)

    def batch_spec(shape):
        return pl.BlockSpec(shape, lambda i: (i, 0))

    def const_spec(shape):
        nd = len(shape)
        if nd == 2:
            return pl.BlockSpec(shape, lambda i: (0, 0))
        return pl.BlockSpec(shape, lambda i: (0, 0, 0))

    in_specs = [
        batch_spec((bt, 13)),
        pl.BlockSpec((_NUM_TABLES, bt), lambda i: (0, i)),
        const_spec((_NUM_TABLES, _K_HASH)),
        const_spec((_NUM_TABLES, _K_HASH)),
        const_spec((_NUM_TABLES, _K_HASH, 256)),
        const_spec((_NUM_TABLES, 256)),
        const_spec((_NUM_TABLES, 256, _EMB_DIM)),
        const_spec((_NUM_PAIRS, 128)),
        const_spec((13, 512)),
        const_spec((1, 512)),
        const_spec((512, 256)),
        const_spec((1, 256)),
        const_spec((256, 64)),
        const_spec((1, 64)),
        const_spec((27 * _EMB_DIM, 512)),
        const_spec((1, 512)),
        const_spec((512, 256)),
        const_spec((1, 256)),
        const_spec((1, 256)),
        const_spec((1, 1)),
    ]
    return pl.pallas_call(
        _fused_body,
        grid=grid,
        in_specs=in_specs,
        out_specs=pl.BlockSpec((bt, 1), lambda i: (i, 0)),
        out_shape=jax.ShapeDtypeStruct((_BATCH, 1), jnp.float32),
        scratch_shapes=[
            pltpu.VMEM((bt, _ZW), jnp.float32),
            pltpu.VMEM((_NUM_TABLES, _K_HASH, 256), jnp.float32),
            pltpu.VMEM((_NUM_TABLES, 256), jnp.float32),
            pltpu.VMEM((_NUM_PAIRS, 512, 128), jnp.float32),
            pltpu.VMEM((_ZW, 512), jnp.float32),
        ],
        interpret=interpret,
    )


def kernel(x_dense, x_indices, hash_a, hash_b, dec_W1, dec_b1, dec_W2,
           dec_b2, bot_W1, bot_b1, bot_W2, bot_b2, bot_W3, bot_b3,
           top_W1, top_b1, top_W2, top_b2, top_W3, top_b3):
    call = _make_call()
    return call(
        x_dense, x_indices, hash_a, hash_b,
        dec_W1, dec_b1, dec_W2, dec_b2.reshape(_NUM_PAIRS, 128),
        bot_W1, bot_b1.reshape(1, -1),
        bot_W2, bot_b2.reshape(1, -1),
        bot_W3, bot_b3.reshape(1, -1),
        top_W1, top_b1.reshape(1, -1),
        top_W2, top_b2.reshape(1, -1),
        top_W3.reshape(1, -1), top_b3.reshape(1, -1),
    )
